# initial kernel scaffold (unmeasured)
import jax
import jax.numpy as jnp
from jax import lax
from jax.experimental import pallas as pl
from jax.experimental.pallas import tpu as pltpu


def kernel(
    x,
):
    def body(*refs):
        pass

    out_shape = jax.ShapeDtypeStruct(..., jnp.float32)
    return pl.pallas_call(body, out_shape=out_shape)(...)


# baseline (device time: 809392 ns/iter reference)
import jax
import jax.numpy as jnp
from jax import lax
from jax.experimental import pallas as pl
from jax.experimental.pallas import tpu as pltpu

M = 32768
N = 1024
H = M // 2
NC = 16
CH = H // NC

EAGER = True


def kernel(x):
    def body(
        x_ref,
        out_ref,
        a_f32,
        a_bf16,
        xrecv,
        in_sems,
        st_sems,
        xsend_sems,
        xrecv_sems,
        ysend_sems,
        yrecv_sems,
    ):
        my_x = lax.axis_index("x")
        my_y = lax.axis_index("y")
        x_nbr = (1 - my_x, my_y)
        y_nbr = (my_x, 1 - my_y)
        row0 = my_y * H
        other0 = (1 - my_y) * H

        def copy_in(c):
            return pltpu.make_async_copy(
                x_ref.at[pl.ds(row0 + c * CH, CH)],
                a_f32.at[c % 2],
                in_sems.at[c],
            )

        def x_rdma(c):
            return pltpu.make_async_remote_copy(
                src_ref=a_bf16.at[c % 2],
                dst_ref=xrecv.at[c],
                send_sem=xsend_sems.at[c],
                recv_sem=xrecv_sems.at[c],
                device_id=x_nbr,
                device_id_type=pl.DeviceIdType.MESH,
            )

        def y_rdma_send(c):
            return pltpu.make_async_remote_copy(
                src_ref=xrecv.at[c],
                dst_ref=out_ref.at[pl.ds(row0 + c * CH, CH)],
                send_sem=ysend_sems.at[c],
                recv_sem=yrecv_sems.at[c],
                device_id=y_nbr,
                device_id_type=pl.DeviceIdType.MESH,
            )

        def y_rdma_recv(c):
            return pltpu.make_async_remote_copy(
                src_ref=xrecv.at[c],
                dst_ref=out_ref.at[pl.ds(other0 + c * CH, CH)],
                send_sem=ysend_sems.at[c],
                recv_sem=yrecv_sems.at[c],
                device_id=y_nbr,
                device_id_type=pl.DeviceIdType.MESH,
            )

        def st_copy(c):
            return pltpu.make_async_copy(
                xrecv.at[c],
                out_ref.at[pl.ds(row0 + c * CH, CH)],
                st_sems.at[c],
            )

        copy_in(0).start()
        copy_in(1).start()

        barrier = pltpu.get_barrier_semaphore()
        for nbr in (x_nbr, y_nbr):
            pl.semaphore_signal(
                barrier, inc=1, device_id=nbr, device_id_type=pl.DeviceIdType.MESH
            )
        pl.semaphore_wait(barrier, 2)

        for c in range(NC):
            copy_in(c).wait()
            if not EAGER and c >= 2:
                x_rdma(c - 2).wait_send()
            a_bf16[c % 2, :, :] = a_f32[c % 2, :, :].astype(jnp.bfloat16)
            if c + 2 < NC:
                copy_in(c + 2).start()

            xr = x_rdma(c)
            xr.start()
            if EAGER:
                xr.wait_send()
            xr.wait_recv()

            xrecv[c, :, :] = xrecv[c, :, :] + a_bf16[c % 2, :, :]

            st = st_copy(c)
            st.start()
            yr = y_rdma_send(c)
            yr.start()
            if EAGER:
                st.wait()
                yr.wait_send()

        for c in range(NC):
            y_rdma_recv(c).wait_recv()

        if EAGER:
            pass
        else:
            for c in (NC - 2, NC - 1):
                x_rdma(c).wait_send()
            for c in range(NC):
                y_rdma_send(c).wait_send()
                st_copy(c).wait()

    return pl.pallas_call(
        body,
        out_shape=jax.ShapeDtypeStruct((M, N), jnp.bfloat16),
        in_specs=[pl.BlockSpec(memory_space=pl.ANY)],
        out_specs=pl.BlockSpec(memory_space=pl.ANY),
        scratch_shapes=[
            pltpu.VMEM((2, CH, N), jnp.float32),
            pltpu.VMEM((2, CH, N), jnp.bfloat16),
            pltpu.VMEM((NC, CH, N), jnp.bfloat16),
            pltpu.SemaphoreType.DMA((NC,)),
            pltpu.SemaphoreType.DMA((NC,)),
            pltpu.SemaphoreType.DMA((NC,)),
            pltpu.SemaphoreType.DMA((NC,)),
            pltpu.SemaphoreType.DMA((NC,)),
            pltpu.SemaphoreType.DMA((NC,)),
        ],
        compiler_params=pltpu.CompilerParams(
            collective_id=0, vmem_limit_bytes=96 * 1024 * 1024
        ),
    )(x)


# device time: 464657 ns/iter; 1.7419x vs baseline; 1.7419x over previous
import jax
import jax.numpy as jnp
from jax import lax
from jax.experimental import pallas as pl
from jax.experimental.pallas import tpu as pltpu

M = 32768
N = 1024
H = M // 2
NC = 16
CH = H // NC

EAGER = False


def kernel(x):
    def body(
        x_ref,
        out_ref,
        a_f32,
        a_bf16,
        xrecv,
        in_sems,
        st_sems,
        xsend_sems,
        xrecv_sems,
        ysend_sems,
        yrecv_sems,
    ):
        my_x = lax.axis_index("x")
        my_y = lax.axis_index("y")
        x_nbr = (1 - my_x, my_y)
        y_nbr = (my_x, 1 - my_y)
        row0 = my_y * H
        other0 = (1 - my_y) * H

        def copy_in(c):
            return pltpu.make_async_copy(
                x_ref.at[pl.ds(row0 + c * CH, CH)],
                a_f32.at[c % 2],
                in_sems.at[c],
            )

        def x_rdma(c):
            return pltpu.make_async_remote_copy(
                src_ref=a_bf16.at[c % 2],
                dst_ref=xrecv.at[c],
                send_sem=xsend_sems.at[c],
                recv_sem=xrecv_sems.at[c],
                device_id=x_nbr,
                device_id_type=pl.DeviceIdType.MESH,
            )

        def y_rdma_send(c):
            return pltpu.make_async_remote_copy(
                src_ref=xrecv.at[c],
                dst_ref=out_ref.at[pl.ds(row0 + c * CH, CH)],
                send_sem=ysend_sems.at[c],
                recv_sem=yrecv_sems.at[c],
                device_id=y_nbr,
                device_id_type=pl.DeviceIdType.MESH,
            )

        def y_rdma_recv(c):
            return pltpu.make_async_remote_copy(
                src_ref=xrecv.at[c],
                dst_ref=out_ref.at[pl.ds(other0 + c * CH, CH)],
                send_sem=ysend_sems.at[c],
                recv_sem=yrecv_sems.at[c],
                device_id=y_nbr,
                device_id_type=pl.DeviceIdType.MESH,
            )

        def st_copy(c):
            return pltpu.make_async_copy(
                xrecv.at[c],
                out_ref.at[pl.ds(row0 + c * CH, CH)],
                st_sems.at[c],
            )

        copy_in(0).start()
        copy_in(1).start()

        barrier = pltpu.get_barrier_semaphore()
        for nbr in (x_nbr, y_nbr):
            pl.semaphore_signal(
                barrier, inc=1, device_id=nbr, device_id_type=pl.DeviceIdType.MESH
            )
        pl.semaphore_wait(barrier, 2)

        for c in range(NC):
            copy_in(c).wait()
            if not EAGER and c >= 2:
                x_rdma(c - 2).wait_send()
            a_bf16[c % 2, :, :] = a_f32[c % 2, :, :].astype(jnp.bfloat16)
            if c + 2 < NC:
                copy_in(c + 2).start()

            xr = x_rdma(c)
            xr.start()
            if EAGER:
                xr.wait_send()
            xr.wait_recv()

            xrecv[c, :, :] = xrecv[c, :, :] + a_bf16[c % 2, :, :]

            st = st_copy(c)
            st.start()
            yr = y_rdma_send(c)
            yr.start()
            if EAGER:
                st.wait()
                yr.wait_send()

        for c in range(NC):
            y_rdma_recv(c).wait_recv()

        if EAGER:
            pass
        else:
            for c in (NC - 2, NC - 1):
                x_rdma(c).wait_send()
            for c in range(NC):
                y_rdma_send(c).wait_send()
                st_copy(c).wait()

    return pl.pallas_call(
        body,
        out_shape=jax.ShapeDtypeStruct((M, N), jnp.bfloat16),
        in_specs=[pl.BlockSpec(memory_space=pl.ANY)],
        out_specs=pl.BlockSpec(memory_space=pl.ANY),
        scratch_shapes=[
            pltpu.VMEM((2, CH, N), jnp.float32),
            pltpu.VMEM((2, CH, N), jnp.bfloat16),
            pltpu.VMEM((NC, CH, N), jnp.bfloat16),
            pltpu.SemaphoreType.DMA((NC,)),
            pltpu.SemaphoreType.DMA((NC,)),
            pltpu.SemaphoreType.DMA((NC,)),
            pltpu.SemaphoreType.DMA((NC,)),
            pltpu.SemaphoreType.DMA((NC,)),
            pltpu.SemaphoreType.DMA((NC,)),
        ],
        compiler_params=pltpu.CompilerParams(
            collective_id=0, vmem_limit_bytes=96 * 1024 * 1024
        ),
    )(x)


# device time: 433692 ns/iter; 1.8663x vs baseline; 1.0714x over previous
import jax
import jax.numpy as jnp
from jax import lax
from jax.experimental import pallas as pl
from jax.experimental.pallas import tpu as pltpu

M = 32768
N = 1024
H = M // 2
NC = 16
CH = H // NC

EAGER = False


def kernel(x):
    def body(
        x_ref,
        out_ref,
        a_f32,
        a_bf16,
        xrecv,
        in_sems,
        st_sems,
        xsend_sems,
        xrecv_sems,
        ysend_sems,
        yrecv_sems,
    ):
        my_x = lax.axis_index("x")
        my_y = lax.axis_index("y")
        x_nbr = (1 - my_x, my_y)
        y_nbr = (my_x, 1 - my_y)
        row0 = my_y * H
        other0 = (1 - my_y) * H

        def copy_in(c):
            return pltpu.make_async_copy(
                x_ref.at[pl.ds(row0 + c * CH, CH)],
                a_f32.at[c % 2],
                in_sems.at[c],
            )

        def x_rdma(c):
            return pltpu.make_async_remote_copy(
                src_ref=a_bf16.at[c % 2],
                dst_ref=xrecv.at[c],
                send_sem=xsend_sems.at[c],
                recv_sem=xrecv_sems.at[c],
                device_id=x_nbr,
                device_id_type=pl.DeviceIdType.MESH,
            )

        def y_rdma_send(c):
            return pltpu.make_async_remote_copy(
                src_ref=xrecv.at[c],
                dst_ref=out_ref.at[pl.ds(row0 + c * CH, CH)],
                send_sem=ysend_sems.at[c],
                recv_sem=yrecv_sems.at[c],
                device_id=y_nbr,
                device_id_type=pl.DeviceIdType.MESH,
            )

        def y_rdma_recv(c):
            return pltpu.make_async_remote_copy(
                src_ref=xrecv.at[c],
                dst_ref=out_ref.at[pl.ds(other0 + c * CH, CH)],
                send_sem=ysend_sems.at[c],
                recv_sem=yrecv_sems.at[c],
                device_id=y_nbr,
                device_id_type=pl.DeviceIdType.MESH,
            )

        def st_copy(c):
            return pltpu.make_async_copy(
                xrecv.at[c],
                out_ref.at[pl.ds(row0 + c * CH, CH)],
                st_sems.at[c],
            )

        copy_in(0).start()
        copy_in(1).start()

        barrier = pltpu.get_barrier_semaphore()
        for nbr in (x_nbr, y_nbr):
            pl.semaphore_signal(
                barrier, inc=1, device_id=nbr, device_id_type=pl.DeviceIdType.MESH
            )
        pl.semaphore_wait(barrier, 2)

        def issue_send(c):
            copy_in(c).wait()
            if c >= 2:
                x_rdma(c - 2).wait_send()
            a_bf16[c % 2, :, :] = a_f32[c % 2, :, :].astype(jnp.bfloat16)
            if c + 2 < NC:
                copy_in(c + 2).start()
            x_rdma(c).start()

        issue_send(0)
        issue_send(1)

        for c in range(NC):
            x_rdma(c).wait_recv()
            xrecv[c, :, :] = xrecv[c, :, :] + a_bf16[c % 2, :, :]

            st = st_copy(c)
            st.start()
            yr = y_rdma_send(c)
            yr.start()
            if EAGER:
                st.wait()
                yr.wait_send()
            if c + 2 < NC:
                issue_send(c + 2)

        for c in range(NC):
            y_rdma_recv(c).wait_recv()

        if EAGER:
            pass
        else:
            for c in (NC - 2, NC - 1):
                x_rdma(c).wait_send()
            for c in range(NC):
                y_rdma_send(c).wait_send()
                st_copy(c).wait()

    return pl.pallas_call(
        body,
        out_shape=jax.ShapeDtypeStruct((M, N), jnp.bfloat16),
        in_specs=[pl.BlockSpec(memory_space=pl.ANY)],
        out_specs=pl.BlockSpec(memory_space=pl.ANY),
        scratch_shapes=[
            pltpu.VMEM((2, CH, N), jnp.float32),
            pltpu.VMEM((2, CH, N), jnp.bfloat16),
            pltpu.VMEM((NC, CH, N), jnp.bfloat16),
            pltpu.SemaphoreType.DMA((NC,)),
            pltpu.SemaphoreType.DMA((NC,)),
            pltpu.SemaphoreType.DMA((NC,)),
            pltpu.SemaphoreType.DMA((NC,)),
            pltpu.SemaphoreType.DMA((NC,)),
            pltpu.SemaphoreType.DMA((NC,)),
        ],
        compiler_params=pltpu.CompilerParams(
            collective_id=0, vmem_limit_bytes=96 * 1024 * 1024
        ),
    )(x)


# device time: 422571 ns/iter; 1.9154x vs baseline; 1.0263x over previous
import jax
import jax.numpy as jnp
from jax import lax
from jax.experimental import pallas as pl
from jax.experimental.pallas import tpu as pltpu

M = 32768
N = 1024
H = M // 2
NC = 32
CH = H // NC

EAGER = False


def kernel(x):
    def body(
        x_ref,
        out_ref,
        a_f32,
        a_bf16,
        xrecv,
        in_sems,
        st_sems,
        xsend_sems,
        xrecv_sems,
        ysend_sems,
        yrecv_sems,
    ):
        my_x = lax.axis_index("x")
        my_y = lax.axis_index("y")
        x_nbr = (1 - my_x, my_y)
        y_nbr = (my_x, 1 - my_y)
        row0 = my_y * H
        other0 = (1 - my_y) * H

        def copy_in(c):
            return pltpu.make_async_copy(
                x_ref.at[pl.ds(row0 + c * CH, CH)],
                a_f32.at[c % 2],
                in_sems.at[c],
            )

        def x_rdma(c):
            return pltpu.make_async_remote_copy(
                src_ref=a_bf16.at[c % 2],
                dst_ref=xrecv.at[c],
                send_sem=xsend_sems.at[c],
                recv_sem=xrecv_sems.at[c],
                device_id=x_nbr,
                device_id_type=pl.DeviceIdType.MESH,
            )

        def y_rdma_send(c):
            return pltpu.make_async_remote_copy(
                src_ref=xrecv.at[c],
                dst_ref=out_ref.at[pl.ds(row0 + c * CH, CH)],
                send_sem=ysend_sems.at[c],
                recv_sem=yrecv_sems.at[c],
                device_id=y_nbr,
                device_id_type=pl.DeviceIdType.MESH,
            )

        def y_rdma_recv(c):
            return pltpu.make_async_remote_copy(
                src_ref=xrecv.at[c],
                dst_ref=out_ref.at[pl.ds(other0 + c * CH, CH)],
                send_sem=ysend_sems.at[c],
                recv_sem=yrecv_sems.at[c],
                device_id=y_nbr,
                device_id_type=pl.DeviceIdType.MESH,
            )

        def st_copy(c):
            return pltpu.make_async_copy(
                xrecv.at[c],
                out_ref.at[pl.ds(row0 + c * CH, CH)],
                st_sems.at[c],
            )

        copy_in(0).start()
        copy_in(1).start()

        barrier = pltpu.get_barrier_semaphore()
        for nbr in (x_nbr, y_nbr):
            pl.semaphore_signal(
                barrier, inc=1, device_id=nbr, device_id_type=pl.DeviceIdType.MESH
            )
        pl.semaphore_wait(barrier, 2)

        def issue_send(c):
            copy_in(c).wait()
            if c >= 2:
                x_rdma(c - 2).wait_send()
            a_bf16[c % 2, :, :] = a_f32[c % 2, :, :].astype(jnp.bfloat16)
            if c + 2 < NC:
                copy_in(c + 2).start()
            x_rdma(c).start()

        issue_send(0)
        issue_send(1)

        for c in range(NC):
            x_rdma(c).wait_recv()
            xrecv[c, :, :] = xrecv[c, :, :] + a_bf16[c % 2, :, :]

            st = st_copy(c)
            st.start()
            yr = y_rdma_send(c)
            yr.start()
            if EAGER:
                st.wait()
                yr.wait_send()
            if c + 2 < NC:
                issue_send(c + 2)

        for c in range(NC):
            y_rdma_recv(c).wait_recv()

        if EAGER:
            pass
        else:
            for c in (NC - 2, NC - 1):
                x_rdma(c).wait_send()
            for c in range(NC):
                y_rdma_send(c).wait_send()
                st_copy(c).wait()

    return pl.pallas_call(
        body,
        out_shape=jax.ShapeDtypeStruct((M, N), jnp.bfloat16),
        in_specs=[pl.BlockSpec(memory_space=pl.ANY)],
        out_specs=pl.BlockSpec(memory_space=pl.ANY),
        scratch_shapes=[
            pltpu.VMEM((2, CH, N), jnp.float32),
            pltpu.VMEM((2, CH, N), jnp.bfloat16),
            pltpu.VMEM((NC, CH, N), jnp.bfloat16),
            pltpu.SemaphoreType.DMA((NC,)),
            pltpu.SemaphoreType.DMA((NC,)),
            pltpu.SemaphoreType.DMA((NC,)),
            pltpu.SemaphoreType.DMA((NC,)),
            pltpu.SemaphoreType.DMA((NC,)),
            pltpu.SemaphoreType.DMA((NC,)),
        ],
        compiler_params=pltpu.CompilerParams(
            collective_id=0, vmem_limit_bytes=96 * 1024 * 1024
        ),
    )(x)
